# baseline (speedup 1.0000x reference)
import functools

import jax
import jax.numpy as jnp
from jax import lax
from jax.experimental import pallas as pl
from jax.experimental.pallas import tpu as pltpu
from jax.experimental.pallas import tpu_sc as plsc

ROWS = 8192
DIM = 1024
NUM_CORES = 2
NUM_SUBCORES = 16
NUM_WORKERS = NUM_CORES * NUM_SUBCORES
ROWS_PER_WORKER = ROWS // NUM_WORKERS
NBUF = 4
CHUNK = 16
NCHUNK = ROWS_PER_WORKER // CHUNK

_mesh = plsc.VectorSubcoreMesh(core_axis_name="c", subcore_axis_name="s")


@functools.partial(
    pl.kernel,
    mesh=_mesh,
    out_type=jax.ShapeDtypeStruct((ROWS, DIM), jnp.float32),
    scratch_types=(
        [pltpu.VMEM((CHUNK, DIM), jnp.float32)] * NBUF
        + [pltpu.SemaphoreType.DMA] * (2 * NBUF)
    ),
)
def _sc_copy(src_hbm, out_hbm, *scratch):
    bufs = scratch[:NBUF]
    in_sems = scratch[NBUF:2 * NBUF]
    out_sems = scratch[2 * NBUF:]

    wid = lax.axis_index("s") * NUM_CORES + lax.axis_index("c")
    base = wid * ROWS_PER_WORKER

    def cp_in(g, b):
        return pltpu.make_async_copy(
            src_hbm.at[pl.ds(base + g * CHUNK, CHUNK)], bufs[b], in_sems[b])

    def cp_out(g, b):
        return pltpu.make_async_copy(
            bufs[b], out_hbm.at[pl.ds(base + g * CHUNK, CHUNK)], out_sems[b])

    for b in range(NBUF):
        cp_in(b, b).start()
    for g in range(NCHUNK):
        b = g % NBUF
        if g >= 1:
            p = g - 1
            nxt = p + NBUF
            if nxt < NCHUNK:
                pb = p % NBUF
                cp_out(p, pb).wait()
                cp_in(nxt, pb).start()
        cp_in(g, b).wait()
        cp_out(g, b).start()
    for g in range(NCHUNK - NBUF, NCHUNK):
        cp_out(g, g % NBUF).wait()


def kernel(x, emb_weight):
    sl = x.shape[1]
    out = _sc_copy(emb_weight)
    return out[:sl]


# speedup vs baseline: 1.0170x; 1.0170x over previous
import functools

import jax
import jax.numpy as jnp
from jax import lax
from jax.experimental import pallas as pl
from jax.experimental.pallas import tpu as pltpu
from jax.experimental.pallas import tpu_sc as plsc

ROWS = 8192
DIM = 1024
NUM_CORES = 2
NUM_SUBCORES = 16
NUM_WORKERS = NUM_CORES * NUM_SUBCORES
ROWS_PER_WORKER = ROWS // NUM_WORKERS
NBUF = 3
CHUNK = 32
NCHUNK = ROWS_PER_WORKER // CHUNK

_mesh = plsc.VectorSubcoreMesh(core_axis_name="c", subcore_axis_name="s")


@functools.partial(
    pl.kernel,
    mesh=_mesh,
    out_type=jax.ShapeDtypeStruct((ROWS, DIM), jnp.float32),
    scratch_types=(
        [pltpu.VMEM((CHUNK, DIM), jnp.float32)] * NBUF
        + [pltpu.SemaphoreType.DMA] * (2 * NBUF)
    ),
)
def _sc_copy(src_hbm, out_hbm, *scratch):
    bufs = scratch[:NBUF]
    in_sems = scratch[NBUF:2 * NBUF]
    out_sems = scratch[2 * NBUF:]

    wid = lax.axis_index("s") * NUM_CORES + lax.axis_index("c")
    base = wid * ROWS_PER_WORKER

    def cp_in(g, b):
        return pltpu.make_async_copy(
            src_hbm.at[pl.ds(base + g * CHUNK, CHUNK)], bufs[b], in_sems[b])

    def cp_out(g, b):
        return pltpu.make_async_copy(
            bufs[b], out_hbm.at[pl.ds(base + g * CHUNK, CHUNK)], out_sems[b])

    for b in range(NBUF):
        cp_in(b, b).start()
    for g in range(NCHUNK):
        b = g % NBUF
        if g >= 1:
            p = g - 1
            nxt = p + NBUF
            if nxt < NCHUNK:
                pb = p % NBUF
                cp_out(p, pb).wait()
                cp_in(nxt, pb).start()
        cp_in(g, b).wait()
        cp_out(g, b).start()
    for g in range(NCHUNK - NBUF, NCHUNK):
        cp_out(g, g % NBUF).wait()


def kernel(x, emb_weight):
    sl = x.shape[1]
    out = _sc_copy(emb_weight)
    return out[:sl]
